# Initial kernel scaffold; baseline (speedup 1.0000x reference)
import jax
import jax.numpy as jnp
from jax.experimental import pallas as pl


def kernel(x, edge_index, Wn, bn, We0, be0, We1, be1):
    raise NotImplementedError("write your pallas kernel here")


# speedup vs baseline: 2.8083x; 2.8083x over previous
import functools

import jax
import jax.numpy as jnp
from jax import lax
from jax.experimental import pallas as pl
from jax.experimental.pallas import tpu as pltpu
from jax.experimental.pallas import tpu_sc as plsc

N_NODES = 10000
N_EDGES = 320000
D = 128
L = 16

NC = 2
NS = 16
NW = NC * NS

CHUNK = 128
N_CHUNKS = N_EDGES // CHUNK
T_STEPS = -(-N_CHUNKS // NW)

_ROWS_TC = 1000
_ROWS_FOLD = 4000


def _node_projections(x, Wn, bn, We0, be0):
    def body(x_ref, wn_ref, bn_ref, we0_ref, be0_ref, p_ref, q_ref):
        h = jnp.maximum(
            jnp.dot(x_ref[...], wn_ref[...], preferred_element_type=jnp.float32)
            + bn_ref[...], 0.0)
        w = we0_ref[...]
        p_ref[...] = jnp.dot(h, w[:D], preferred_element_type=jnp.float32) + be0_ref[...]
        q_ref[...] = jnp.dot(h, w[D:], preferred_element_type=jnp.float32)

    return pl.pallas_call(
        body,
        grid=(N_NODES // _ROWS_TC,),
        in_specs=[
            pl.BlockSpec((_ROWS_TC, D), lambda i: (i, 0)),
            pl.BlockSpec((D, D), lambda i: (0, 0)),
            pl.BlockSpec((1, D), lambda i: (0, 0)),
            pl.BlockSpec((2 * D, D), lambda i: (0, 0)),
            pl.BlockSpec((1, D), lambda i: (0, 0)),
        ],
        out_specs=[
            pl.BlockSpec((_ROWS_TC, D), lambda i: (i, 0)),
            pl.BlockSpec((_ROWS_TC, D), lambda i: (i, 0)),
        ],
        out_shape=[
            jax.ShapeDtypeStruct((N_NODES, D), jnp.float32),
            jax.ShapeDtypeStruct((N_NODES, D), jnp.float32),
        ],
    )(x, Wn, bn.reshape(1, D), We0, be0.reshape(1, D))


def _edge_partials(P, Q, src2d, dst2d, w1):
    mesh = plsc.VectorSubcoreMesh(
        core_axis_name="c", subcore_axis_name="s", num_cores=NC, num_subcores=NS)

    @functools.partial(
        pl.kernel,
        mesh=mesh,
        out_type=jax.ShapeDtypeStruct((N_EDGES, L), jnp.float32),
        scratch_types=[
            pltpu.VMEM((CHUNK,), jnp.int32),
            pltpu.VMEM((CHUNK,), jnp.int32),
            pltpu.VMEM((CHUNK, D), jnp.float32),
            pltpu.VMEM((CHUNK, D), jnp.float32),
            pltpu.VMEM((CHUNK, L), jnp.float32),
            pltpu.VMEM((D,), jnp.float32),
            pltpu.SemaphoreType.DMA,
            pltpu.SemaphoreType.DMA,
        ],
    )
    def k(p_hbm, q_hbm, src_hbm, dst_hbm, w1_hbm, out_hbm,
          idx_s, idx_d, bufP, bufQ, out_v, w1_v, semP, semQ):
        wid = lax.axis_index("s") * NC + lax.axis_index("c")
        pltpu.sync_copy(w1_hbm, w1_v)
        w1r = [w1_v[pl.ds(c * L, L)] for c in range(D // L)]

        def step(t, carry):
            cid = t * NW + wid

            @pl.when(cid < N_CHUNKS)
            def _():
                pltpu.sync_copy(src_hbm.at[cid], idx_s)
                pltpu.sync_copy(dst_hbm.at[cid], idx_d)
                cp_p = pltpu.async_copy(p_hbm.at[idx_s], bufP, semP)
                cp_q = pltpu.async_copy(q_hbm.at[idx_d], bufQ, semQ)
                cp_p.wait()
                cp_q.wait()

                def edge(e, c2):
                    acc = jnp.zeros((L,), jnp.float32)
                    for c in range(D // L):
                        pv = bufP[e, pl.ds(c * L, L)]
                        qv = bufQ[e, pl.ds(c * L, L)]
                        acc = acc + jnp.maximum(pv + qv, 0.0) * w1r[c]
                    out_v[e, :] = acc
                    return c2

                lax.fori_loop(0, CHUNK, edge, 0, unroll=4)
                pltpu.sync_copy(out_v, out_hbm.at[pl.ds(cid * CHUNK, CHUNK)])

            return carry

        lax.fori_loop(0, T_STEPS, step, 0)

    return k(P, Q, src2d, dst2d, w1)


def _fold_partials(part8, be1):
    def body(g_ref, fold_ref, be1_ref, s_ref):
        s_ref[...] = jnp.dot(
            g_ref[...], fold_ref[...], preferred_element_type=jnp.float32
        ) + be1_ref[...]

    rows = N_EDGES // 8
    fold = jnp.asarray(
        (jnp.arange(D)[:, None] // L) == jnp.arange(8)[None, :], jnp.float32)
    return pl.pallas_call(
        body,
        grid=(rows // _ROWS_FOLD,),
        in_specs=[
            pl.BlockSpec((_ROWS_FOLD, D), lambda i: (i, 0)),
            pl.BlockSpec((D, 8), lambda i: (0, 0)),
            pl.BlockSpec((1, 8), lambda i: (0, 0)),
        ],
        out_specs=pl.BlockSpec((_ROWS_FOLD, 8), lambda i: (i, 0)),
        out_shape=jax.ShapeDtypeStruct((rows, 8), jnp.float32),
    )(part8, fold, jnp.broadcast_to(be1.reshape(1, 1), (1, 8)))


def kernel(x, edge_index, Wn, bn, We0, be0, We1, be1):
    P, Q = _node_projections(x, Wn, bn, We0, be0)
    ei = edge_index.astype(jnp.int32)
    src2d = ei[0].reshape(N_CHUNKS, CHUNK)
    dst2d = ei[1].reshape(N_CHUNKS, CHUNK)
    w1 = We1[:, 0]
    part = _edge_partials(P, Q, src2d, dst2d, w1)
    part8 = part.reshape(N_EDGES // 8, D)
    score = _fold_partials(part8, be1)
    return score.reshape(N_EDGES, 1)


# speedup vs baseline: 7.0329x; 2.5043x over previous
import functools

import jax
import jax.numpy as jnp
from jax import lax
from jax.experimental import pallas as pl
from jax.experimental.pallas import tpu as pltpu
from jax.experimental.pallas import tpu_sc as plsc

N_NODES = 10000
N_EDGES = 320000
D = 128
L = 16

NC = 2
NS = 16
NW = NC * NS

CHUNK = 128
N_CHUNKS = N_EDGES // CHUNK
N_CHUNKS_PAD = 2528
OCTETS = N_CHUNKS_PAD // 8
IMAX = 8 * (-(-OCTETS // NW))
TPAIR = IMAX // 2

_ROWS_TC = 2000
_ROWS_FOLD = 8000


def _node_projections(x, Wn, bn, We0, be0):
    def body(x_ref, wn_ref, bn_ref, we0_ref, be0_ref, p_ref, q_ref):
        h = jnp.maximum(
            jnp.dot(x_ref[...], wn_ref[...], preferred_element_type=jnp.float32)
            + bn_ref[...], 0.0)
        w = we0_ref[...]
        p_ref[...] = jnp.dot(h, w[:D], preferred_element_type=jnp.float32) + be0_ref[...]
        q_ref[...] = jnp.dot(h, w[D:], preferred_element_type=jnp.float32)

    return pl.pallas_call(
        body,
        grid=(N_NODES // _ROWS_TC,),
        in_specs=[
            pl.BlockSpec((_ROWS_TC, D), lambda i: (i, 0)),
            pl.BlockSpec((D, D), lambda i: (0, 0)),
            pl.BlockSpec((1, D), lambda i: (0, 0)),
            pl.BlockSpec((2 * D, D), lambda i: (0, 0)),
            pl.BlockSpec((1, D), lambda i: (0, 0)),
        ],
        out_specs=[
            pl.BlockSpec((_ROWS_TC, D), lambda i: (i, 0)),
            pl.BlockSpec((_ROWS_TC, D), lambda i: (i, 0)),
        ],
        out_shape=[
            jax.ShapeDtypeStruct((N_NODES, D), jnp.float32),
            jax.ShapeDtypeStruct((N_NODES, D), jnp.float32),
        ],
    )(x, Wn, bn.reshape(1, D), We0, be0.reshape(1, D))


def _edge_partials(P, Q, src2d, dst2d, w1, be1v):
    mesh = plsc.VectorSubcoreMesh(
        core_axis_name="c", subcore_axis_name="s", num_cores=NC, num_subcores=NS)

    @functools.partial(
        pl.kernel,
        mesh=mesh,
        out_type=jax.ShapeDtypeStruct((N_EDGES // 8, D), jnp.float32),
        scratch_types=[
            pltpu.VMEM((IMAX, CHUNK), jnp.int32),
            pltpu.VMEM((IMAX, CHUNK), jnp.int32),
            [pltpu.VMEM((CHUNK, D), jnp.float32)] * 2,
            [pltpu.VMEM((CHUNK, D), jnp.float32)] * 2,
            [pltpu.VMEM((CHUNK // 8, D), jnp.float32)] * 2,
            pltpu.VMEM((D,), jnp.float32),
            pltpu.VMEM((L,), jnp.float32),
            [pltpu.SemaphoreType.DMA] * 2,
            [pltpu.SemaphoreType.DMA] * 2,
            [pltpu.SemaphoreType.DMA] * 2,
        ],
    )
    def k(p_hbm, q_hbm, src_hbm, dst_hbm, w1_hbm, be1_hbm, out_hbm,
          idx_s2, idx_d2, bufP, bufQ, out_v, w1_v, be1_v, semP, semQ, semO):
        wid = lax.axis_index("s") * NC + lax.axis_index("c")
        ostart = wid * OCTETS // NW
        start = pl.multiple_of(ostart * 8, 8)
        nw = ((wid + 1) * OCTETS // NW - ostart) * 8

        pltpu.sync_copy(w1_hbm, w1_v)
        pltpu.sync_copy(be1_hbm, be1_v)
        pltpu.sync_copy(src_hbm.at[pl.ds(start, IMAX)], idx_s2)
        pltpu.sync_copy(dst_hbm.at[pl.ds(start, IMAX)], idx_d2)
        w1r = [w1_v[pl.ds(c * L, L)] for c in range(D // L)]
        be1r = be1_v[...]

        def issue(r, b):
            pltpu.async_copy(p_hbm.at[idx_s2.at[r]], bufP[b], semP[b])
            pltpu.async_copy(q_hbm.at[idx_d2.at[r]], bufQ[b], semQ[b])

        def wait_gather(r, b):
            pltpu.make_async_copy(p_hbm.at[idx_s2.at[r]], bufP[b], semP[b]).wait()
            pltpu.make_async_copy(q_hbm.at[idx_d2.at[r]], bufQ[b], semQ[b]).wait()

        issue(0, 0)

        def pair(t, carry):
            for b in range(2):
                r = t * 2 + b

                @pl.when((r + 1 < nw) & (start + r + 1 < N_CHUNKS))
                def _():
                    issue(r + 1, 1 - b)

                @pl.when((r < nw) & (start + r < N_CHUNKS))
                def _():
                    wait_gather(r, b)

                    @pl.when(r >= 2)
                    def _():
                        pltpu.make_async_copy(
                            p_hbm.at[pl.ds(0, CHUNK // 8)], out_v[b], semO[b]
                        ).wait()

                    bP, bQ, oV = bufP[b], bufQ[b], out_v[b]

                    def edge_row(row, c2):
                        for kk in range(8):
                            e = row * 8 + kk
                            acc = be1r
                            for c in range(D // L):
                                pv = bP[e, pl.ds(c * L, L)]
                                qv = bQ[e, pl.ds(c * L, L)]
                                acc = acc + jnp.maximum(pv + qv, 0.0) * w1r[c]
                            oV[row, pl.ds(kk * L, L)] = acc
                        return c2

                    lax.fori_loop(0, CHUNK // 8, edge_row, 0)
                    pltpu.async_copy(
                        oV, out_hbm.at[pl.ds((start + r) * (CHUNK // 8), CHUNK // 8)],
                        semO[b])
            return carry

        lax.fori_loop(0, TPAIR, pair, 0)
        for b in range(2):
            pltpu.make_async_copy(
                p_hbm.at[pl.ds(0, CHUNK // 8)], out_v[b], semO[b]).wait()

    return k(P, Q, src2d, dst2d, w1, be1v)


def _fold_partials(part8):
    def body(g_ref, s_ref):
        fold = jnp.asarray(
            lax.broadcasted_iota(jnp.int32, (D, 8), 0) // L
            == lax.broadcasted_iota(jnp.int32, (D, 8), 1), jnp.float32)
        s_ref[...] = jnp.dot(g_ref[...], fold, preferred_element_type=jnp.float32)

    rows = N_EDGES // 8
    return pl.pallas_call(
        body,
        grid=(rows // _ROWS_FOLD,),
        in_specs=[pl.BlockSpec((_ROWS_FOLD, D), lambda i: (i, 0))],
        out_specs=pl.BlockSpec((_ROWS_FOLD, 8), lambda i: (i, 0)),
        out_shape=jax.ShapeDtypeStruct((rows, 8), jnp.float32),
    )(part8)


def kernel(x, edge_index, Wn, bn, We0, be0, We1, be1):
    P, Q = _node_projections(x, Wn, bn, We0, be0)
    ei = edge_index.astype(jnp.int32)
    pad = ((0, N_CHUNKS_PAD - N_CHUNKS), (0, 0))
    src2d = jnp.pad(ei[0].reshape(N_CHUNKS, CHUNK), pad)
    dst2d = jnp.pad(ei[1].reshape(N_CHUNKS, CHUNK), pad)
    w1 = We1[:, 0]
    be1v = jnp.full((L,), be1[0] / L, dtype=jnp.float32)
    part8 = _edge_partials(P, Q, src2d, dst2d, w1, be1v)
    score = _fold_partials(part8)
    return score.reshape(N_EDGES, 1)


# speedup vs baseline: 7.9121x; 1.1250x over previous
import functools

import jax
import jax.numpy as jnp
from jax import lax
from jax.experimental import pallas as pl
from jax.experimental.pallas import tpu as pltpu
from jax.experimental.pallas import tpu_sc as plsc

N_NODES = 10000
N_EDGES = 320000
D = 128
L = 16

NC = 2
NS = 16
NW = NC * NS

CHUNK = 128
N_CHUNKS = N_EDGES // CHUNK
N_CHUNKS_PAD = 2528
OCTETS = N_CHUNKS_PAD // 8
IMAX = 8 * (-(-OCTETS // NW))
TPAIR = IMAX // 2
ROW = CHUNK * L

_ROWS_TC = 2000
_CHUNKS_FOLD = 250


def _node_projections(x, Wn, bn, We0, be0):
    def body(x_ref, wn_ref, bn_ref, we0_ref, be0_ref, p_ref, q_ref):
        h = jnp.maximum(
            jnp.dot(x_ref[...], wn_ref[...], preferred_element_type=jnp.float32)
            + bn_ref[...], 0.0)
        w = we0_ref[...]
        p_ref[...] = jnp.dot(h, w[:D], preferred_element_type=jnp.float32) + be0_ref[...]
        q_ref[...] = jnp.dot(h, w[D:], preferred_element_type=jnp.float32)

    return pl.pallas_call(
        body,
        grid=(N_NODES // _ROWS_TC,),
        in_specs=[
            pl.BlockSpec((_ROWS_TC, D), lambda i: (i, 0)),
            pl.BlockSpec((D, D), lambda i: (0, 0)),
            pl.BlockSpec((1, D), lambda i: (0, 0)),
            pl.BlockSpec((2 * D, D), lambda i: (0, 0)),
            pl.BlockSpec((1, D), lambda i: (0, 0)),
        ],
        out_specs=[
            pl.BlockSpec((_ROWS_TC, D), lambda i: (i, 0)),
            pl.BlockSpec((_ROWS_TC, D), lambda i: (i, 0)),
        ],
        out_shape=[
            jax.ShapeDtypeStruct((N_NODES, D), jnp.float32),
            jax.ShapeDtypeStruct((N_NODES, D), jnp.float32),
        ],
    )(x, Wn, bn.reshape(1, D), We0, be0.reshape(1, D))


def _edge_partials(P, Q, epad, w1, be1v):
    mesh = plsc.VectorSubcoreMesh(
        core_axis_name="c", subcore_axis_name="s", num_cores=NC, num_subcores=NS)

    @functools.partial(
        pl.kernel,
        mesh=mesh,
        out_type=jax.ShapeDtypeStruct((N_CHUNKS, ROW), jnp.float32),
        scratch_types=[
            pltpu.VMEM((IMAX * CHUNK,), jnp.int32),
            pltpu.VMEM((IMAX * CHUNK,), jnp.int32),
            [pltpu.VMEM((CHUNK, D), jnp.float32)] * 2,
            [pltpu.VMEM((CHUNK, D), jnp.float32)] * 2,
            [pltpu.VMEM((ROW,), jnp.float32)] * 2,
            pltpu.VMEM((D,), jnp.float32),
            pltpu.VMEM((L,), jnp.float32),
            [pltpu.SemaphoreType.DMA] * 2,
            [pltpu.SemaphoreType.DMA] * 2,
            [pltpu.SemaphoreType.DMA] * 2,
        ],
    )
    def k(p_hbm, q_hbm, e_hbm, w1_hbm, be1_hbm, out_hbm,
          idx_s, idx_d, bufP, bufQ, out_v, w1_v, be1_v, semP, semQ, semO):
        wid = lax.axis_index("s") * NC + lax.axis_index("c")
        ostart = wid * OCTETS // NW
        start = pl.multiple_of(ostart * 8, 8)
        nw = ((wid + 1) * OCTETS // NW - ostart) * 8

        pltpu.sync_copy(w1_hbm, w1_v)
        pltpu.sync_copy(be1_hbm, be1_v)
        pltpu.sync_copy(e_hbm.at[0, pl.ds(start * CHUNK, IMAX * CHUNK)], idx_s)
        pltpu.sync_copy(e_hbm.at[1, pl.ds(start * CHUNK, IMAX * CHUNK)], idx_d)
        w1r = [w1_v[pl.ds(c * L, L)] for c in range(D // L)]
        be1r = be1_v[...]

        def issue(r, b):
            pltpu.async_copy(
                p_hbm.at[idx_s.at[pl.ds(r * CHUNK, CHUNK)]], bufP[b], semP[b])
            pltpu.async_copy(
                q_hbm.at[idx_d.at[pl.ds(r * CHUNK, CHUNK)]], bufQ[b], semQ[b])

        def wait_gather(r, b):
            pltpu.make_async_copy(
                p_hbm.at[idx_s.at[pl.ds(r * CHUNK, CHUNK)]], bufP[b], semP[b]).wait()
            pltpu.make_async_copy(
                q_hbm.at[idx_d.at[pl.ds(r * CHUNK, CHUNK)]], bufQ[b], semQ[b]).wait()

        issue(0, 0)

        def pair(t, carry):
            for b in range(2):
                r = t * 2 + b

                @pl.when((r + 1 < nw) & (start + r + 1 < N_CHUNKS))
                def _():
                    issue(r + 1, 1 - b)

                @pl.when((r < nw) & (start + r < N_CHUNKS))
                def _():
                    wait_gather(r, b)

                    @pl.when(r >= 2)
                    def _():
                        pltpu.make_async_copy(
                            out_hbm.at[0], out_v[b], semO[b]).wait()

                    bP, bQ, oV = bufP[b], bufQ[b], out_v[b]

                    def edge_row(row, c2):
                        for kk in range(8):
                            e = row * 8 + kk
                            acc = be1r
                            for c in range(D // L):
                                pv = bP[e, pl.ds(c * L, L)]
                                qv = bQ[e, pl.ds(c * L, L)]
                                acc = acc + jnp.maximum(pv + qv, 0.0) * w1r[c]
                            oV[pl.ds(e * L, L)] = acc
                        return c2

                    lax.fori_loop(0, CHUNK // 8, edge_row, 0)
                    pltpu.async_copy(oV, out_hbm.at[start + r], semO[b])
            return carry

        lax.fori_loop(0, TPAIR, pair, 0)
        for b in range(2):
            pltpu.make_async_copy(out_hbm.at[0], out_v[b], semO[b]).wait()

    return k(P, Q, epad, w1, be1v)


def _fold_partials(part):
    def body(g_ref, s_ref):
        fold = jnp.asarray(
            lax.broadcasted_iota(jnp.int32, (ROW, D), 0) // L
            == lax.broadcasted_iota(jnp.int32, (ROW, D), 1), jnp.float32)
        s_ref[...] = jnp.dot(g_ref[...], fold, preferred_element_type=jnp.float32)

    return pl.pallas_call(
        body,
        grid=(1,),
        in_specs=[pl.BlockSpec((N_CHUNKS, ROW), lambda i: (0, 0))],
        out_specs=pl.BlockSpec((N_CHUNKS, D), lambda i: (0, 0)),
        out_shape=jax.ShapeDtypeStruct((N_CHUNKS, D), jnp.float32),
    )(part)


def kernel(x, edge_index, Wn, bn, We0, be0, We1, be1):
    P, Q = _node_projections(x, Wn, bn, We0, be0)
    ei = edge_index.astype(jnp.int32)
    epad = jnp.pad(ei, ((0, 0), (0, N_CHUNKS_PAD * CHUNK - N_EDGES)))
    w1 = We1[:, 0]
    be1v = jnp.full((L,), be1[0] / L, dtype=jnp.float32)
    part = _edge_partials(P, Q, epad, w1, be1v)
    score = _fold_partials(part)
    return score.reshape(N_EDGES, 1)
